# baseline (speedup 1.0000x reference)
import jax
import jax.numpy as jnp
from jax import lax
from jax.experimental import pallas as pl
from jax.experimental.pallas import tpu as pltpu
from jax.experimental.pallas import tpu_sc as plsc

_NUM_PRE = 100000
_EMBED_DIM = 64
_BATCH = 16384

_L = 16
_NC, _NS = 2, 16
_NW = _NC * _NS
_BPW = _BATCH // _NW
_G = _BPW + 32
_NROWS = 2 * _G
_OUT_ROWS = _BATCH + _NW * _NROWS


def _body(idx_hbm, w_pre_hbm, w_new_hbm, out_hbm,
          idx_stage, idx_lo, idx_hi, pos_buf, rows_v, sem):
    wid = lax.axis_index("s") * _NC + lax.axis_index("c")
    base = wid * _BPW
    dump0 = _BATCH + wid * _NROWS
    lane = lax.iota(jnp.int32, _L)

    pltpu.sync_copy(idx_hbm.at[pl.ds(base, _BPW)], idx_stage)

    def init_idx(i, _):
        idx_lo[pl.ds(i * _L, _L)] = jnp.zeros((_L,), jnp.int32)
        idx_hi[pl.ds(i * _L, _L)] = jnp.zeros((_L,), jnp.int32)
        return 0
    lax.fori_loop(0, _G // _L, init_idx, 0)

    def init_pos(i, _):
        pos_buf[pl.ds(i * _L, _L)] = dump0 + i * _L + lane
        return 0
    lax.fori_loop(0, _NROWS // _L, init_pos, 0)

    def compact_body(r, carry):
        o_lo, o_hi = carry
        v = idx_stage[pl.ds(r * _L, _L)]
        pos = base + r * _L + lane
        m_lo = v < _NUM_PRE
        m_hi = jnp.logical_not(m_lo)
        plsc.store_compressed(idx_lo.at[pl.ds(o_lo, _L)], v, mask=m_lo)
        plsc.store_compressed(pos_buf.at[pl.ds(o_lo, _L)], pos, mask=m_lo)
        plsc.store_compressed(idx_hi.at[pl.ds(o_hi - _G, _L)], v - _NUM_PRE,
                              mask=m_hi)
        plsc.store_compressed(pos_buf.at[pl.ds(o_hi, _L)], pos, mask=m_hi)
        c = jnp.sum(jnp.where(m_lo, 1, 0).astype(jnp.int32))
        return o_lo + c, o_hi + (_L - c)
    lax.fori_loop(0, _BPW // _L, compact_body,
                  (jnp.int32(0), jnp.int32(_G)))

    cp_lo = pltpu.async_copy(w_pre_hbm.at[idx_lo],
                             rows_v.at[pl.ds(0, _G), :], sem)
    cp_hi = pltpu.async_copy(w_new_hbm.at[idx_hi],
                             rows_v.at[pl.ds(_G, _G), :], sem)
    cp_lo.wait()
    cp_hi.wait()
    pltpu.sync_copy(rows_v, out_hbm.at[pos_buf])


@jax.jit
def _combined_lookup(indices, w_pre, w_new):
    run = pl.kernel(
        _body,
        out_type=jax.ShapeDtypeStruct((_OUT_ROWS, _EMBED_DIM), jnp.float32),
        mesh=plsc.VectorSubcoreMesh(core_axis_name="c", subcore_axis_name="s",
                                    num_cores=_NC, num_subcores=_NS),
        scratch_types=[
            pltpu.VMEM((_BPW,), jnp.int32),
            pltpu.VMEM((_G,), jnp.int32),
            pltpu.VMEM((_G,), jnp.int32),
            pltpu.VMEM((_NROWS,), jnp.int32),
            pltpu.VMEM((_NROWS, _EMBED_DIM), jnp.float32),
            pltpu.SemaphoreType.DMA,
        ],
        compiler_params=pltpu.CompilerParams(use_tc_tiling_on_sc=False,
                                             needs_layout_passes=False),
    )
    return run(indices, w_pre, w_new)


def kernel(indices, W_pre, W_new):
    out = _combined_lookup(indices.astype(jnp.int32), W_pre, W_new)
    return out[:_BATCH]


# speedup vs baseline: 2.0333x; 2.0333x over previous
import functools

import jax
import jax.numpy as jnp
from jax import lax
from jax.experimental import pallas as pl
from jax.experimental.pallas import tpu as pltpu
from jax.experimental.pallas import tpu_sc as plsc

_NUM_PRE = 100000
_EMBED_DIM = 64
_BATCH = 16384

_L = 16
_NC, _NS = 2, 16
_NW = _NC * _NS
_BPW = _BATCH // _NW
_C = 64
_NBUF = _BPW + 2 * _C
_DUMP = _BATCH


def _body(idx_hbm, w_pre_hbm, w_new_hbm, out_hbm,
          idx_stage, idx_buf, pos_buf, rows_v, sem):
    wid = lax.axis_index("s") * _NC + lax.axis_index("c")
    base = wid * _BPW

    pltpu.sync_copy(idx_hbm.at[pl.ds(base, _BPW)], idx_stage)

    def init_body(i, _):
        idx_buf[pl.ds(i * _L, _L)] = jnp.zeros((_L,), jnp.int32)
        pos_buf[pl.ds(i * _L, _L)] = jnp.full((_L,), _DUMP, jnp.int32)
        return 0
    lax.fori_loop(0, _NBUF // _L, init_body, 0)

    def count_body(r, cntv):
        v = idx_stage[pl.ds(r * _L, _L)]
        return cntv + jnp.where(v < _NUM_PRE, 1, 0).astype(jnp.int32)
    cnt_v = lax.fori_loop(0, _BPW // _L, count_body,
                          jnp.zeros((_L,), jnp.int32))
    n_lo = jnp.sum(cnt_v)
    n_hi = _BPW - n_lo
    n_lo_pad = ((n_lo + _C - 1) // _C) * _C

    lane = lax.iota(jnp.int32, _L)

    def compact_body(r, carry):
        o_lo, o_hi = carry
        v = idx_stage[pl.ds(r * _L, _L)]
        pos = base + r * _L + lane
        m_lo = v < _NUM_PRE
        m_hi = jnp.logical_not(m_lo)
        plsc.store_compressed(idx_buf.at[pl.ds(o_lo, _L)], v, mask=m_lo)
        plsc.store_compressed(pos_buf.at[pl.ds(o_lo, _L)], pos, mask=m_lo)
        plsc.store_compressed(idx_buf.at[pl.ds(o_hi, _L)], v - _NUM_PRE,
                              mask=m_hi)
        plsc.store_compressed(pos_buf.at[pl.ds(o_hi, _L)], pos, mask=m_hi)
        c = jnp.sum(jnp.where(m_lo, 1, 0).astype(jnp.int32))
        return o_lo + c, o_hi + (_L - c)
    lax.fori_loop(0, _BPW // _L, compact_body, (jnp.int32(0), n_lo_pad))

    n_lo_ch = n_lo_pad // _C
    n_hi_ch = (n_hi + _C - 1) // _C

    def fire_lo(j, _):
        pltpu.async_copy(w_pre_hbm.at[idx_buf.at[pl.ds(j * _C, _C)]],
                         rows_v.at[pl.ds(j * _C, _C), :], sem)
        return 0
    lax.fori_loop(0, n_lo_ch, fire_lo, 0)

    def fire_hi(j, _):
        off = n_lo_pad + j * _C
        pltpu.async_copy(w_new_hbm.at[idx_buf.at[pl.ds(off, _C)]],
                         rows_v.at[pl.ds(off, _C), :], sem)
        return 0
    lax.fori_loop(0, n_hi_ch, fire_hi, 0)

    def drain(j, _):
        pltpu.make_async_copy(w_pre_hbm.at[idx_buf.at[pl.ds(0, _C)]],
                              rows_v.at[pl.ds(0, _C), :], sem).wait()
        return 0
    lax.fori_loop(0, n_lo_ch + n_hi_ch, drain, 0)

    pltpu.sync_copy(rows_v, out_hbm.at[pos_buf])


@jax.jit
def _combined_lookup(indices, w_pre, w_new):
    run = pl.kernel(
        _body,
        out_type=jax.ShapeDtypeStruct((_BATCH + 1, _EMBED_DIM), jnp.float32),
        mesh=plsc.VectorSubcoreMesh(core_axis_name="c", subcore_axis_name="s",
                                    num_cores=_NC, num_subcores=_NS),
        scratch_types=[
            pltpu.VMEM((_BPW,), jnp.int32),
            pltpu.VMEM((_NBUF,), jnp.int32),
            pltpu.VMEM((_NBUF,), jnp.int32),
            pltpu.VMEM((_NBUF, _EMBED_DIM), jnp.float32),
            pltpu.SemaphoreType.DMA,
        ],
        compiler_params=pltpu.CompilerParams(use_tc_tiling_on_sc=False,
                                             needs_layout_passes=False),
    )
    return run(indices, w_pre, w_new)


def kernel(indices, W_pre, W_new):
    out = _combined_lookup(indices.astype(jnp.int32), W_pre, W_new)
    return out[:_BATCH]


# speedup vs baseline: 2.7741x; 1.3643x over previous
import functools

import jax
import jax.numpy as jnp
from jax import lax
from jax.experimental import pallas as pl
from jax.experimental.pallas import tpu as pltpu
from jax.experimental.pallas import tpu_sc as plsc

_NUM_PRE = 100000
_EMBED_DIM = 64
_BATCH = 16384

_L = 16
_NC, _NS = 2, 16
_NW = _NC * _NS
_BPW = _BATCH // _NW
_C = 16
_NBUF = _BPW + 2 * _C
_DUMP = _BATCH


def _body(idx_hbm, w_pre_hbm, w_new_hbm, out_hbm,
          idx_stage, idx_buf, pos_buf, rows_v, sem):
    wid = lax.axis_index("s") * _NC + lax.axis_index("c")
    base = wid * _BPW

    pltpu.sync_copy(idx_hbm.at[pl.ds(base, _BPW)], idx_stage)

    def init_body(i, _):
        idx_buf[pl.ds(i * _L, _L)] = jnp.zeros((_L,), jnp.int32)
        pos_buf[pl.ds(i * _L, _L)] = jnp.full((_L,), _DUMP, jnp.int32)
        return 0
    lax.fori_loop(0, _NBUF // _L, init_body, 0)

    def count_body(r, cntv):
        v = idx_stage[pl.ds(r * _L, _L)]
        return cntv + jnp.where(v < _NUM_PRE, 1, 0).astype(jnp.int32)
    cnt_v = lax.fori_loop(0, _BPW // _L, count_body,
                          jnp.zeros((_L,), jnp.int32))
    n_lo = jnp.sum(cnt_v)
    n_hi = _BPW - n_lo
    n_lo_pad = ((n_lo + _C - 1) // _C) * _C

    lane = lax.iota(jnp.int32, _L)

    def compact_body(r, carry):
        o_lo, o_hi = carry
        v = idx_stage[pl.ds(r * _L, _L)]
        pos = base + r * _L + lane
        m_lo = v < _NUM_PRE
        m_hi = jnp.logical_not(m_lo)
        plsc.store_compressed(idx_buf.at[pl.ds(o_lo, _L)], v, mask=m_lo)
        plsc.store_compressed(pos_buf.at[pl.ds(o_lo, _L)], pos, mask=m_lo)
        plsc.store_compressed(idx_buf.at[pl.ds(o_hi, _L)], v - _NUM_PRE,
                              mask=m_hi)
        plsc.store_compressed(pos_buf.at[pl.ds(o_hi, _L)], pos, mask=m_hi)
        c = jnp.sum(jnp.where(m_lo, 1, 0).astype(jnp.int32))
        return o_lo + c, o_hi + (_L - c)
    lax.fori_loop(0, _BPW // _L, compact_body, (jnp.int32(0), n_lo_pad))

    n_lo_ch = n_lo_pad // _C
    n_hi_ch = (n_hi + _C - 1) // _C

    def fire_lo(j, _):
        pltpu.async_copy(w_pre_hbm.at[idx_buf.at[pl.ds(j * _C, _C)]],
                         rows_v.at[pl.ds(j * _C, _C), :], sem)
        return 0
    lax.fori_loop(0, n_lo_ch, fire_lo, 0)

    def fire_hi(j, _):
        off = n_lo_pad + j * _C
        pltpu.async_copy(w_new_hbm.at[idx_buf.at[pl.ds(off, _C)]],
                         rows_v.at[pl.ds(off, _C), :], sem)
        return 0
    lax.fori_loop(0, n_hi_ch, fire_hi, 0)

    def drain(j, _):
        pltpu.make_async_copy(w_pre_hbm.at[idx_buf.at[pl.ds(0, _C)]],
                              rows_v.at[pl.ds(0, _C), :], sem).wait()
        return 0
    lax.fori_loop(0, n_lo_ch + n_hi_ch, drain, 0)

    pltpu.sync_copy(rows_v, out_hbm.at[pos_buf])


@jax.jit
def _combined_lookup(indices, w_pre, w_new):
    run = pl.kernel(
        _body,
        out_type=jax.ShapeDtypeStruct((_BATCH + 1, _EMBED_DIM), jnp.float32),
        mesh=plsc.VectorSubcoreMesh(core_axis_name="c", subcore_axis_name="s",
                                    num_cores=_NC, num_subcores=_NS),
        scratch_types=[
            pltpu.VMEM((_BPW,), jnp.int32),
            pltpu.VMEM((_NBUF,), jnp.int32),
            pltpu.VMEM((_NBUF,), jnp.int32),
            pltpu.VMEM((_NBUF, _EMBED_DIM), jnp.float32),
            pltpu.SemaphoreType.DMA,
        ],
        compiler_params=pltpu.CompilerParams(use_tc_tiling_on_sc=False,
                                             needs_layout_passes=False),
    )
    return run(indices, w_pre, w_new)


def kernel(indices, W_pre, W_new):
    out = _combined_lookup(indices.astype(jnp.int32), W_pre, W_new)
    return out[:_BATCH]
